# baseline (speedup 1.0000x reference)
import functools
import math

import jax
import jax.numpy as jnp
from jax import lax
from jax.experimental import pallas as pl
from jax.experimental.pallas import tpu as pltpu
from jax.experimental.pallas import tpu_sc as plsc

NC, NS, L = 2, 16, 16
NW = NC * NS
TL = 128


def _make_sc_kernel(B, C, H, W, K, PAIRS, PAD):
    PPW = PAD // NW
    CCH = C // L

    mesh = plsc.VectorSubcoreMesh(core_axis_name="c", subcore_axis_name="s")

    @functools.partial(
        pl.kernel,
        out_type=jax.ShapeDtypeStruct((NW, 2, L), jnp.float32),
        mesh=mesh,
        compiler_params=pltpu.CompilerParams(needs_layout_passes=False),
        scratch_types=[
            pltpu.VMEM((PPW,), jnp.int32),
            pltpu.VMEM((PPW,), jnp.int32),
            pltpu.VMEM((PPW,), jnp.int32),
            pltpu.VMEM((PPW,), jnp.float32),
            pltpu.VMEM((PPW,), jnp.int32),
            pltpu.VMEM((4, C), jnp.int32),
            pltpu.VMEM((4, C, TL), jnp.float32),
            pltpu.VMEM((PPW, C), jnp.float32),
            pltpu.VMEM((PPW, C), jnp.float32),
            pltpu.VMEM((2, L), jnp.float32),
            pltpu.SemaphoreType.DMA,
            pltpu.SemaphoreType.DMA,
        ],
    )
    def sc_kernel(out4d_hbm, rb_hbm, wt_hbm, wm_hbm, maskf_hbm, rows_hbm,
                  tgt_hbm, out_hbm, rb_v, wt_v, wm_v, maskf_v, rows_v,
                  idx_v, win_v, gath_v, tgt_v, part_v, sem0, sem1):
        rowmap_hbm = out4d_hbm.reshape(B * C * H, W)
        wid = lax.axis_index("s") * NC + lax.axis_index("c")
        lane = jnp.arange(L, dtype=jnp.int32)

        pltpu.sync_copy(rb_hbm.at[wid], rb_v)
        pltpu.sync_copy(wt_hbm.at[wid], wt_v)
        pltpu.sync_copy(wm_hbm.at[wid], wm_v)
        pltpu.sync_copy(maskf_hbm.at[wid], maskf_v)
        pltpu.sync_copy(rows_hbm.at[wid], rows_v)

        tgt_cp = pltpu.async_copy(tgt_hbm.at[rows_v], tgt_v, sem1)

        def _scalar_at(ref, j):
            chunk = ref[pl.ds((j // L) * L, L)]
            sel = jnp.where(lane == j % L, chunk, 0)
            return lax.reduce_sum_p.bind(sel, axes=(0,))

        coffs = [(lane + cc * L) * H for cc in range(CCH)]

        def _fire(j):
            par = j % 4
            rbs = plsc.load_gather(rb_v, [jnp.full((L,), j, jnp.int32)])
            for cc in range(CCH):
                idx_v[par, pl.ds(cc * L, L)] = rbs + coffs[cc]
            wt_s = pl.multiple_of(_scalar_at(wt_v, j) * TL, TL)
            pltpu.make_async_copy(
                rowmap_hbm.at[idx_v.at[par], pl.ds(wt_s, TL)],
                win_v.at[par], sem0).start()

        def _wait_one():
            pltpu.make_async_copy(
                rowmap_hbm.at[pl.ds(0, C), pl.ds(0, TL)],
                win_v.at[0], sem0).wait()

        def _extract(j):
            par = j % 4
            wmf = jnp.full((L,), 1, jnp.int32) * _scalar_at(wm_v, j)
            pars = jnp.full((L,), par, jnp.int32)
            for cc in range(CCH):
                gath_v[j, pl.ds(cc * L, L)] = plsc.load_gather(
                    win_v, [pars, cc * L + lane, wmf])

        def pipe_body(j, carry):
            _wait_one()
            _extract(j)

            @pl.when(j + 4 < PPW)
            def _():
                _fire(j + 4)

            return carry

        for j0 in range(4):
            _fire(j0)
        lax.fori_loop(0, PPW, pipe_body, 0)
        tgt_cp.wait()

        def mse_body(j, acc):
            mf = plsc.load_gather(maskf_v, [jnp.full((L,), j, jnp.int32)])
            for cc in range(CCH):
                d = gath_v[j, pl.ds(cc * L, L)] - tgt_v[j, pl.ds(cc * L, L)]
                acc = acc + d * d * mf
            return acc

        acc = lax.fori_loop(0, PPW, mse_body, jnp.zeros((L,), jnp.float32))

        cnt = jnp.zeros((L,), jnp.float32)
        for jj in range(PPW // L):
            cnt = cnt + maskf_v[pl.ds(jj * L, L)]

        part_v[0, :] = acc
        part_v[1, :] = cnt
        pltpu.sync_copy(part_v, out_hbm.at[wid])

    return sc_kernel


def kernel(output, mask, ind, target):
    B, C, H, W = output.shape
    K = ind.shape[1]
    PAIRS = B * K
    PAD = ((PAIRS + 8 * NW - 1) // (8 * NW)) * (8 * NW)

    tgt2d = target.reshape(PAIRS, C)

    p = jnp.arange(PAD, dtype=jnp.int32)
    valid = p < PAIRS
    psafe = jnp.minimum(p, PAIRS - 1)
    ind_flat = ind.reshape(-1).astype(jnp.int32)[psafe]
    hh = jnp.where(valid, ind_flat // W, 0)
    ww = jnp.where(valid, ind_flat % W, 0)
    b_of_p = jnp.minimum(psafe // K, B - 1)
    rb = jnp.where(valid, b_of_p * (C * H) + hh, 0).reshape(NW, -1)
    wt = (ww // TL).reshape(NW, -1)
    wm = (ww % TL).reshape(NW, -1)
    maskf = (mask.reshape(-1) > 0).astype(jnp.float32)[psafe]
    maskf = jnp.where(valid, maskf, 0.0).reshape(NW, -1)
    rows = psafe.reshape(NW, -1)

    sck = _make_sc_kernel(B, C, H, W, K, PAIRS, PAD)
    parts = sck(output, rb, wt, wm, maskf, rows, tgt2d)

    sumsq = jnp.sum(parts[:, 0, :])
    cnt = jnp.sum(parts[:, 1, :])
    denom = jnp.maximum(cnt * C, 1.0)
    return jnp.where(cnt > 0, sumsq / denom, jnp.asarray(0.0, jnp.float32))


# speedup vs baseline: 1.2109x; 1.2109x over previous
import functools
import math

import jax
import jax.numpy as jnp
from jax import lax
from jax.experimental import pallas as pl
from jax.experimental.pallas import tpu as pltpu
from jax.experimental.pallas import tpu_sc as plsc

NC, NS, L = 2, 16, 16
NW = NC * NS
TL = 128
DEPTH = 4


def _make_sc_kernel(B, C, H, W, K, PAIRS, PAD):
    PPW = PAD // NW
    CCH = C // L
    JCH = PPW // L

    mesh = plsc.VectorSubcoreMesh(core_axis_name="c", subcore_axis_name="s")

    @functools.partial(
        pl.kernel,
        out_type=jax.ShapeDtypeStruct((NW, 2, L), jnp.float32),
        mesh=mesh,
        compiler_params=pltpu.CompilerParams(needs_layout_passes=False),
        scratch_types=[
            pltpu.VMEM((PPW,), jnp.int32),
            pltpu.VMEM((PPW,), jnp.int32),
            pltpu.VMEM((PPW,), jnp.float32),
            pltpu.VMEM((PPW,), jnp.int32),
            pltpu.VMEM((DEPTH, C), jnp.int32),
            pltpu.VMEM((DEPTH, C, TL), jnp.float32),
            pltpu.VMEM((PPW, C), jnp.float32),
            pltpu.VMEM((PPW, C), jnp.float32),
            pltpu.VMEM((2, L), jnp.float32),
            pltpu.SemaphoreType.DMA,
            pltpu.SemaphoreType.DMA,
        ],
    )
    def sc_kernel(out4d_hbm, ind_hbm, maski_hbm, tgt_hbm, out_hbm,
                  ind_v, maski_v, maskf_v, rows_v, idx_v, win_v, gath_v,
                  tgt_v, part_v, sem0, sem1):
        rowmap_hbm = out4d_hbm.reshape(B * C * H, W)
        wid = lax.axis_index("s") * NC + lax.axis_index("c")
        lane = jnp.arange(L, dtype=jnp.int32)
        wbase = wid * PPW

        pltpu.sync_copy(ind_hbm.at[wid], ind_v)
        pltpu.sync_copy(maski_hbm.at[wid], maski_v)

        for jj in range(JCH):
            rows_v[pl.ds(jj * L, L)] = jnp.minimum(
                wbase + jj * L + lane, PAIRS - 1)
            maskf_v[pl.ds(jj * L, L)] = maski_v[pl.ds(jj * L, L)].astype(
                jnp.float32)

        def zero_body(j, carry):
            for cc in range(CCH):
                gath_v[j, pl.ds(cc * L, L)] = jnp.zeros((L,), jnp.float32)
            return carry

        lax.fori_loop(0, PPW, zero_body, 0)

        tgt_cp = pltpu.async_copy(tgt_hbm.at[rows_v], tgt_v, sem1)

        def _scalar_at(ref, j):
            chunk = ref[pl.ds((j // L) * L, L)]
            sel = jnp.where(lane == j % L, chunk, 0)
            return lax.reduce_sum_p.bind(sel, axes=(0,))

        coffs = [(lane + cc * L) * H for cc in range(CCH)]

        def _masked(j):
            return _scalar_at(maski_v, jnp.minimum(j, PPW - 1)) > 0

        def _fire(j):
            par = j % DEPTH
            p = wbase + j
            ind_s = _scalar_at(ind_v, j)
            b_s = jnp.minimum(p // K, B - 1)
            h_s = ind_s // W
            w_s = ind_s - h_s * W
            rb_s = b_s * (C * H) + h_s
            rbs = jnp.full((L,), rb_s, jnp.int32)
            for cc in range(CCH):
                idx_v[par, pl.ds(cc * L, L)] = rbs + coffs[cc]
            wt_s = pl.multiple_of((w_s // TL) * TL, TL)
            pltpu.make_async_copy(
                rowmap_hbm.at[idx_v.at[par], pl.ds(wt_s, TL)],
                win_v.at[par], sem0).start()

        def _wait_one():
            pltpu.make_async_copy(
                rowmap_hbm.at[pl.ds(0, C), pl.ds(0, TL)],
                win_v.at[0], sem0).wait()

        def _extract(j):
            par = j % DEPTH
            ind_s = _scalar_at(ind_v, j)
            wm_s = (ind_s - (ind_s // W) * W) % TL
            wmf = jnp.full((L,), wm_s, jnp.int32)
            pars = jnp.full((L,), par, jnp.int32)
            for cc in range(CCH):
                gath_v[j, pl.ds(cc * L, L)] = plsc.load_gather(
                    win_v, [pars, cc * L + lane, wmf])

        def pipe_body(j, carry):
            @pl.when(_masked(j))
            def _():
                _wait_one()
                _extract(j)

            @pl.when(jnp.logical_and(j + DEPTH < PPW, _masked(j + DEPTH)))
            def _():
                _fire(j + DEPTH)

            return carry

        for j0 in range(DEPTH):
            @pl.when(_masked(j0))
            def _():
                _fire(j0)

        lax.fori_loop(0, PPW, pipe_body, 0)
        tgt_cp.wait()

        def mse_body(j, acc):
            mf = plsc.load_gather(maskf_v, [jnp.full((L,), j, jnp.int32)])
            for cc in range(CCH):
                d = gath_v[j, pl.ds(cc * L, L)] - tgt_v[j, pl.ds(cc * L, L)]
                acc = acc + d * d * mf
            return acc

        acc = lax.fori_loop(0, PPW, mse_body, jnp.zeros((L,), jnp.float32))

        cnt = jnp.zeros((L,), jnp.float32)
        for jj in range(JCH):
            cnt = cnt + maskf_v[pl.ds(jj * L, L)]

        part_v[0, :] = acc
        part_v[1, :] = cnt
        pltpu.sync_copy(part_v, out_hbm.at[wid])

    return sc_kernel


def kernel(output, mask, ind, target):
    B, C, H, W = output.shape
    K = ind.shape[1]
    PAIRS = B * K
    PAD = ((PAIRS + 8 * NW - 1) // (8 * NW)) * (8 * NW)

    tgt2d = target.reshape(PAIRS, C)
    ind_pad = jnp.pad(ind.reshape(-1).astype(jnp.int32),
                      (0, PAD - PAIRS)).reshape(NW, -1)
    mask_pad = jnp.pad((mask.reshape(-1) > 0).astype(jnp.int32),
                       (0, PAD - PAIRS)).reshape(NW, -1)

    sck = _make_sc_kernel(B, C, H, W, K, PAIRS, PAD)
    parts = sck(output, ind_pad, mask_pad, tgt2d)

    sumsq = jnp.sum(parts[:, 0, :])
    cnt = jnp.sum(parts[:, 1, :])
    denom = jnp.maximum(cnt * C, 1.0)
    return jnp.where(cnt > 0, sumsq / denom, jnp.asarray(0.0, jnp.float32))


# speedup vs baseline: 1.2377x; 1.0221x over previous
import functools
import math

import jax
import jax.numpy as jnp
from jax import lax
from jax.experimental import pallas as pl
from jax.experimental.pallas import tpu as pltpu
from jax.experimental.pallas import tpu_sc as plsc

NC, NS, L = 2, 16, 16
NW = NC * NS
TL = 128
DEPTH = 4


def _make_sc_kernel(B, C, H, W, K):
    QW = NW // B
    PPW = K // QW
    TWIN = ((PPW + 7) // 8) * 8 + 8
    CCH = C // L
    JCH = (PPW + L - 1) // L

    mesh = plsc.VectorSubcoreMesh(core_axis_name="c", subcore_axis_name="s")

    @functools.partial(
        pl.kernel,
        out_type=jax.ShapeDtypeStruct((NW, 2, L), jnp.float32),
        mesh=mesh,
        compiler_params=pltpu.CompilerParams(needs_layout_passes=False),
        scratch_types=[
            pltpu.VMEM((K,), jnp.int32),
            pltpu.VMEM((K,), jnp.int32),
            pltpu.VMEM((DEPTH, C), jnp.int32),
            pltpu.VMEM((DEPTH, C, TL), jnp.float32),
            pltpu.VMEM((PPW + 3, C), jnp.float32),
            pltpu.VMEM((TWIN, C), jnp.float32),
            pltpu.VMEM((2, L), jnp.float32),
            pltpu.SemaphoreType.DMA,
            pltpu.SemaphoreType.DMA,
        ],
    )
    def sc_kernel(out4d_hbm, ind_hbm, mask_hbm, tgt_hbm, out_hbm,
                  ind_v, maski_v, idx_v, win_v, gath_v, tgt_v, part_v,
                  sem0, sem1):
        rowmap_hbm = out4d_hbm.reshape(B * C * H, W)
        wid = lax.axis_index("s") * NC + lax.axis_index("c")
        lane = jnp.arange(L, dtype=jnp.int32)
        b_s = wid // QW
        k0 = (wid % QW) * PPW
        k8 = pl.multiple_of((k0 // 8) * 8, 8)
        skew = k0 - k8

        pltpu.sync_copy(ind_hbm.at[b_s], ind_v)
        pltpu.sync_copy(mask_hbm.at[b_s], maski_v)

        tgt_cp = pltpu.async_copy(
            tgt_hbm.at[b_s, pl.ds(k8, TWIN), :], tgt_v, sem1)

        def _gat1(ref, j):
            return plsc.load_gather(ref, [jnp.full((L,), j, jnp.int32)])

        def _scalar_at(ref, j):
            sel = jnp.where(lane == 0, _gat1(ref, j), 0)
            return lax.reduce_sum_p.bind(sel, axes=(0,))

        def zero_body(j, carry):
            for cc in range(CCH):
                gath_v[j, pl.ds(cc * L, L)] = jnp.zeros((L,), jnp.float32)
            return carry

        lax.fori_loop(0, PPW, zero_body, 0)

        coffs = [(lane + cc * L) * H for cc in range(CCH)]

        def _masked(j):
            return _scalar_at(maski_v, jnp.minimum(k0 + j, K - 1)) > 0

        def _fire(j):
            par = j % DEPTH
            ind_s = _scalar_at(ind_v, k0 + j)
            h_s = ind_s // W
            rb_s = b_s * (C * H) + h_s
            rbs = jnp.full((L,), rb_s, jnp.int32)
            for cc in range(CCH):
                idx_v[par, pl.ds(cc * L, L)] = rbs + coffs[cc]
            w_s = ind_s - h_s * W
            wt_s = pl.multiple_of((w_s // TL) * TL, TL)
            pltpu.make_async_copy(
                rowmap_hbm.at[idx_v.at[par], pl.ds(wt_s, TL)],
                win_v.at[par], sem0).start()

        def _wait_one():
            pltpu.make_async_copy(
                rowmap_hbm.at[pl.ds(0, C), pl.ds(0, TL)],
                win_v.at[0], sem0).wait()

        def _extract(j):
            par = j % DEPTH
            ind_s = _scalar_at(ind_v, k0 + j)
            wm_s = (ind_s - (ind_s // W) * W) % TL
            wmf = jnp.full((L,), wm_s, jnp.int32)
            pars = jnp.full((L,), par, jnp.int32)
            for cc in range(CCH):
                gath_v[j, pl.ds(cc * L, L)] = plsc.load_gather(
                    win_v, [pars, cc * L + lane, wmf])

        def pipe_body(j, carry):
            @pl.when(_masked(j))
            def _():
                _wait_one()
                _extract(j)

            @pl.when(jnp.logical_and(j + DEPTH < PPW, _masked(j + DEPTH)))
            def _():
                _fire(j + DEPTH)

            return carry

        for j0 in range(DEPTH):
            @pl.when(_masked(j0))
            def _():
                _fire(j0)

        lax.fori_loop(0, PPW, pipe_body, 0)
        tgt_cp.wait()

        def mse_body(j, acc):
            mi = _gat1(maski_v, k0 + j)
            mf = jnp.where(mi > 0, 1.0, 0.0).astype(jnp.float32)
            for cc in range(CCH):
                d = (gath_v[j, pl.ds(cc * L, L)]
                     - tgt_v[skew + j, pl.ds(cc * L, L)])
                acc = acc + d * d * mf
            return acc

        acc = lax.fori_loop(0, PPW, mse_body, jnp.zeros((L,), jnp.float32))

        cnt = jnp.zeros((L,), jnp.float32)
        for jj in range(JCH):
            kk = k0 + jj * L + lane
            mi = plsc.load_gather(maski_v, [jnp.minimum(kk, K - 1)])
            ok = jnp.logical_and(kk < k0 + PPW, mi > 0)
            cnt = cnt + jnp.where(ok, 1.0, 0.0).astype(jnp.float32)

        part_v[0, :] = acc
        part_v[1, :] = cnt
        pltpu.sync_copy(part_v, out_hbm.at[wid])

    return sc_kernel


def kernel(output, mask, ind, target):
    B, C, H, W = output.shape
    K = ind.shape[1]

    sck = _make_sc_kernel(B, C, H, W, K)
    parts = sck(output, ind.astype(jnp.int32), mask.astype(jnp.int32),
                target)

    sumsq = jnp.sum(parts[:, 0, :])
    cnt = jnp.sum(parts[:, 1, :])
    denom = jnp.maximum(cnt * C, 1.0)
    return jnp.where(cnt > 0, sumsq / denom, jnp.asarray(0.0, jnp.float32))


# speedup vs baseline: 12.0243x; 9.7154x over previous
import functools
import math

import jax
import jax.numpy as jnp
from jax import lax
from jax.experimental import pallas as pl
from jax.experimental.pallas import tpu as pltpu
from jax.experimental.pallas import tpu_sc as plsc

NC, NS, L = 2, 16, 16
NW = NC * NS


def _make_sc_kernel(B, C, HW, K):
    QW = NW // B
    PPW = K // QW
    PPWP = ((PPW + L - 1) // L) * L
    TWIN = ((PPW + 7) // 8) * 8 + 8
    CCH = C // L
    JCH = PPWP // L

    mesh = plsc.VectorSubcoreMesh(core_axis_name="c", subcore_axis_name="s")

    @functools.partial(
        pl.kernel,
        out_type=jax.ShapeDtypeStruct((NW, 2, L), jnp.float32),
        mesh=mesh,
        compiler_params=pltpu.CompilerParams(needs_layout_passes=False),
        scratch_types=[
            pltpu.VMEM((K,), jnp.int32),
            pltpu.VMEM((K,), jnp.int32),
            pltpu.VMEM((PPWP,), jnp.int32),
            pltpu.VMEM((PPWP, C), jnp.float32),
            pltpu.VMEM((TWIN, C), jnp.float32),
            pltpu.VMEM((2, L), jnp.float32),
            pltpu.SemaphoreType.DMA,
            pltpu.SemaphoreType.DMA,
        ],
    )
    def sc_kernel(feat_hbm, ind_hbm, mask_hbm, tgt_hbm, out_hbm,
                  ind_v, maski_v, rows_v, gath_v, tgt_v, part_v,
                  sem0, sem1):
        wid = lax.axis_index("s") * NC + lax.axis_index("c")
        lane = jnp.arange(L, dtype=jnp.int32)
        b_s = wid // QW
        k0 = (wid % QW) * PPW
        k8 = pl.multiple_of((k0 // 8) * 8, 8)
        skew = k0 - k8

        pltpu.sync_copy(ind_hbm.at[b_s], ind_v)
        pltpu.sync_copy(mask_hbm.at[b_s], maski_v)

        tgt_cp = pltpu.async_copy(
            tgt_hbm.at[b_s, pl.ds(k8, TWIN), :], tgt_v, sem1)

        bbase = jnp.full((L,), b_s * HW, jnp.int32)
        for jj in range(JCH):
            kk = jnp.minimum(k0 + jj * L + lane, K - 1)
            rows_v[pl.ds(jj * L, L)] = bbase + plsc.load_gather(ind_v, [kk])

        pltpu.async_copy(feat_hbm.at[rows_v], gath_v, sem0).wait()
        tgt_cp.wait()

        def _gat1(ref, j):
            return plsc.load_gather(ref, [jnp.full((L,), j, jnp.int32)])

        def mse_body(j, acc):
            mi = _gat1(maski_v, k0 + j)
            mf = jnp.where(mi > 0, 1.0, 0.0).astype(jnp.float32)
            for cc in range(CCH):
                d = (gath_v[j, pl.ds(cc * L, L)]
                     - tgt_v[skew + j, pl.ds(cc * L, L)])
                acc = acc + d * d * mf
            return acc

        acc = lax.fori_loop(0, PPW, mse_body, jnp.zeros((L,), jnp.float32))

        cnt = jnp.zeros((L,), jnp.float32)
        for jj in range(JCH):
            kk = k0 + jj * L + lane
            mi = plsc.load_gather(maski_v, [jnp.minimum(kk, K - 1)])
            ok = jnp.logical_and(kk < k0 + PPW, mi > 0)
            cnt = cnt + jnp.where(ok, 1.0, 0.0).astype(jnp.float32)

        part_v[0, :] = acc
        part_v[1, :] = cnt
        pltpu.sync_copy(part_v, out_hbm.at[wid])

    return sc_kernel


def kernel(output, mask, ind, target):
    B, C, H, W = output.shape
    K = ind.shape[1]
    HW = H * W

    feat = jnp.transpose(output, (0, 2, 3, 1)).reshape(B * HW, C)

    sck = _make_sc_kernel(B, C, HW, K)
    parts = sck(feat, ind.astype(jnp.int32), mask.astype(jnp.int32), target)

    sumsq = jnp.sum(parts[:, 0, :])
    cnt = jnp.sum(parts[:, 1, :])
    denom = jnp.maximum(cnt * C, 1.0)
    return jnp.where(cnt > 0, sumsq / denom, jnp.asarray(0.0, jnp.float32))


# speedup vs baseline: 12.0689x; 1.0037x over previous
import functools
import math

import jax
import jax.numpy as jnp
from jax import lax
from jax.experimental import pallas as pl
from jax.experimental.pallas import tpu as pltpu
from jax.experimental.pallas import tpu_sc as plsc

NC, NS, L = 2, 16, 16
NW = NC * NS


def _make_sc_kernel(B, C, HW, K):
    QW = NW // B
    PPW = K // QW
    PPWP = ((PPW + L - 1) // L) * L
    TWIN = ((PPW + 7) // 8) * 8 + 8
    CCH = C // L
    JCH = PPWP // L

    mesh = plsc.VectorSubcoreMesh(core_axis_name="c", subcore_axis_name="s")

    @functools.partial(
        pl.kernel,
        out_type=jax.ShapeDtypeStruct((NW, 2, L), jnp.float32),
        mesh=mesh,
        compiler_params=pltpu.CompilerParams(needs_layout_passes=False),
        scratch_types=[
            pltpu.VMEM((K,), jnp.int32),
            pltpu.VMEM((K,), jnp.int32),
            pltpu.VMEM((PPWP,), jnp.float32),
            pltpu.VMEM((PPWP,), jnp.int32),
            pltpu.VMEM((PPWP, C), jnp.float32),
            pltpu.VMEM((TWIN, C), jnp.float32),
            pltpu.VMEM((2, L), jnp.float32),
            pltpu.SemaphoreType.DMA,
            pltpu.SemaphoreType.DMA,
        ],
    )
    def sc_kernel(feat_hbm, ind_hbm, mask_hbm, tgt_hbm, out_hbm,
                  ind_v, maski_v, maskf_v, rows_v, gath_v, tgt_v, part_v,
                  sem0, sem1):
        wid = lax.axis_index("s") * NC + lax.axis_index("c")
        lane = jnp.arange(L, dtype=jnp.int32)
        b_s = wid // QW
        k0 = (wid % QW) * PPW
        k8 = pl.multiple_of((k0 // 8) * 8, 8)
        skew = k0 - k8

        pltpu.sync_copy(ind_hbm.at[b_s], ind_v)
        pltpu.sync_copy(mask_hbm.at[b_s], maski_v)

        tgt_cp = pltpu.async_copy(
            tgt_hbm.at[b_s, pl.ds(k8, TWIN), :], tgt_v, sem1)

        bbase = jnp.full((L,), b_s * HW, jnp.int32)
        cnt = jnp.zeros((L,), jnp.float32)
        for jj in range(JCH):
            kk = k0 + jj * L + lane
            kks = jnp.minimum(kk, K - 1)
            rows_v[pl.ds(jj * L, L)] = bbase + plsc.load_gather(ind_v, [kks])
            mi = plsc.load_gather(maski_v, [kks])
            ok = jnp.logical_and(kk < k0 + PPW, mi > 0)
            mfv = jnp.where(ok, 1.0, 0.0).astype(jnp.float32)
            maskf_v[pl.ds(jj * L, L)] = mfv
            cnt = cnt + mfv

        pltpu.async_copy(feat_hbm.at[rows_v], gath_v, sem0).wait()
        tgt_cp.wait()

        def _gat1(ref, j):
            return plsc.load_gather(ref, [jnp.full((L,), j, jnp.int32)])

        def mse_body(j, acc):
            mf = _gat1(maskf_v, j)
            s = jnp.zeros((L,), jnp.float32)
            for cc in range(CCH):
                d = (gath_v[j, pl.ds(cc * L, L)]
                     - tgt_v[skew + j, pl.ds(cc * L, L)])
                s = s + d * d
            return acc + s * mf

        acc = lax.fori_loop(0, PPW, mse_body, jnp.zeros((L,), jnp.float32))

        part_v[0, :] = acc
        part_v[1, :] = cnt
        pltpu.sync_copy(part_v, out_hbm.at[wid])

    return sc_kernel


def kernel(output, mask, ind, target):
    B, C, H, W = output.shape
    K = ind.shape[1]
    HW = H * W

    feat = jnp.transpose(output, (0, 2, 3, 1)).reshape(B * HW, C)

    sck = _make_sc_kernel(B, C, HW, K)
    parts = sck(feat, ind.astype(jnp.int32), mask.astype(jnp.int32), target)

    sumsq = jnp.sum(parts[:, 0, :])
    cnt = jnp.sum(parts[:, 1, :])
    denom = jnp.maximum(cnt * C, 1.0)
    return jnp.where(cnt > 0, sumsq / denom, jnp.asarray(0.0, jnp.float32))


# speedup vs baseline: 12.2573x; 1.0156x over previous
import functools
import math

import jax
import jax.numpy as jnp
from jax import lax
from jax.experimental import pallas as pl
from jax.experimental.pallas import tpu as pltpu
from jax.experimental.pallas import tpu_sc as plsc

NC, NS, L = 2, 16, 16
NW = NC * NS


def _make_sc_kernel(B, C, HW, K):
    QW = NW // B
    PPW = K // QW
    PPWP = ((PPW + L - 1) // L) * L
    CCH = C // L
    JCH = PPWP // L

    mesh = plsc.VectorSubcoreMesh(core_axis_name="c", subcore_axis_name="s")

    @functools.partial(
        pl.kernel,
        out_type=jax.ShapeDtypeStruct((NW, 2, L), jnp.float32),
        mesh=mesh,
        compiler_params=pltpu.CompilerParams(needs_layout_passes=False),
        scratch_types=[
            pltpu.VMEM((K,), jnp.int32),
            pltpu.VMEM((K,), jnp.int32),
            pltpu.VMEM((PPWP,), jnp.float32),
            pltpu.VMEM((PPWP,), jnp.int32),
            pltpu.VMEM((PPWP,), jnp.int32),
            pltpu.VMEM((PPWP, C), jnp.float32),
            pltpu.VMEM((PPWP, C), jnp.float32),
            pltpu.VMEM((2, L), jnp.float32),
            pltpu.SemaphoreType.DMA,
            pltpu.SemaphoreType.DMA,
        ],
    )
    def sc_kernel(feat_hbm, ind_hbm, mask_hbm, tgtT_hbm, out_hbm,
                  ind_v, maski_v, maskf_v, rows_v, trows_v, gath_v, tgt_v,
                  part_v, sem0, sem1):
        tgt2d_hbm = tgtT_hbm.reshape(K * B, C)
        wid = lax.axis_index("s") * NC + lax.axis_index("c")
        lane = jnp.arange(L, dtype=jnp.int32)
        b_s = wid // QW
        k0 = (wid % QW) * PPW

        pltpu.sync_copy(ind_hbm.at[b_s], ind_v)
        pltpu.sync_copy(mask_hbm.at[b_s], maski_v)

        bbase = jnp.full((L,), b_s * HW, jnp.int32)
        cnt = jnp.zeros((L,), jnp.float32)
        for jj in range(JCH):
            kk = k0 + jj * L + lane
            kks = jnp.minimum(kk, K - 1)
            rows_v[pl.ds(jj * L, L)] = bbase + plsc.load_gather(ind_v, [kks])
            trows_v[pl.ds(jj * L, L)] = kks * B + b_s
            mi = plsc.load_gather(maski_v, [kks])
            ok = jnp.logical_and(kk < k0 + PPW, mi > 0)
            mfv = jnp.where(ok, 1.0, 0.0).astype(jnp.float32)
            maskf_v[pl.ds(jj * L, L)] = mfv
            cnt = cnt + mfv

        tgt_cp = pltpu.async_copy(tgt2d_hbm.at[trows_v], tgt_v, sem1)
        pltpu.async_copy(feat_hbm.at[rows_v], gath_v, sem0).wait()
        tgt_cp.wait()

        def _gat1(ref, j):
            return plsc.load_gather(ref, [jnp.full((L,), j, jnp.int32)])

        def mse_body(j, acc):
            mf = _gat1(maskf_v, j)
            s = jnp.zeros((L,), jnp.float32)
            for cc in range(CCH):
                d = (gath_v[j, pl.ds(cc * L, L)]
                     - tgt_v[j, pl.ds(cc * L, L)])
                s = s + d * d
            return acc + s * mf

        acc = lax.fori_loop(0, PPW, mse_body, jnp.zeros((L,), jnp.float32))

        part_v[0, :] = acc
        part_v[1, :] = cnt
        pltpu.sync_copy(part_v, out_hbm.at[wid])

    return sc_kernel


def kernel(output, mask, ind, target):
    B, C, H, W = output.shape
    K = ind.shape[1]
    HW = H * W

    feat = jnp.transpose(output, (0, 2, 3, 1)).reshape(B * HW, C)
    tgtT = jnp.transpose(target, (1, 0, 2))

    sck = _make_sc_kernel(B, C, HW, K)
    parts = sck(feat, ind.astype(jnp.int32), mask.astype(jnp.int32), tgtT)

    sumsq = jnp.sum(parts[:, 0, :])
    cnt = jnp.sum(parts[:, 1, :])
    denom = jnp.maximum(cnt * C, 1.0)
    return jnp.where(cnt > 0, sumsq / denom, jnp.asarray(0.0, jnp.float32))
